# Initial kernel scaffold; baseline (speedup 1.0000x reference)
import math

import jax
import jax.numpy as jnp
from jax.experimental import pallas as pl
from jax.experimental.pallas import tpu as pltpu


def _moe_body(idx_ref, hw_ref, x_ref, W1_ref, b1_ref, W2_ref, b2_ref, out_ref):
    e = pl.program_id(1)
    x = x_ref[...]
    h = jnp.dot(x, W1_ref[0], preferred_element_type=jnp.float32) + b1_ref[0]
    h = 0.5 * h * (1.0 + jax.lax.erf(h * (1.0 / math.sqrt(2.0))))
    y = jnp.dot(h, W2_ref[0], preferred_element_type=jnp.float32) + b2_ref[0]
    w = jnp.sum(jnp.where(idx_ref[...] == e, hw_ref[...], 0.0), axis=1)
    contrib = w[:, None] * y

    @pl.when(e == 0)
    def _():
        out_ref[...] = contrib

    @pl.when(e != 0)
    def _():
        out_ref[...] += contrib


def kernel(x_modality, expert_indices, hard_weights, W1, b1, W2, b2):
    B, D = x_modality.shape
    E, _, H = W1.shape
    K = expert_indices.shape[1]
    idx = expert_indices.astype(jnp.int32)

    BM = 512
    nb = B // BM
    grid = (nb, E)

    return pl.pallas_call(
        _moe_body,
        grid=grid,
        in_specs=[
            pl.BlockSpec((BM, K), lambda i, e: (i, 0)),
            pl.BlockSpec((BM, K), lambda i, e: (i, 0)),
            pl.BlockSpec((BM, D), lambda i, e: (i, 0)),
            pl.BlockSpec((1, D, H), lambda i, e: (e, 0, 0)),
            pl.BlockSpec((1, H), lambda i, e: (e, 0)),
            pl.BlockSpec((1, H, D), lambda i, e: (e, 0, 0)),
            pl.BlockSpec((1, D), lambda i, e: (e, 0)),
        ],
        out_specs=pl.BlockSpec((BM, D), lambda i, e: (i, 0)),
        out_shape=jax.ShapeDtypeStruct((B, D), jnp.float32),
        compiler_params=pltpu.CompilerParams(
            dimension_semantics=("parallel", "arbitrary"),
        ),
    )(idx, hard_weights, x_modality, W1, b1, W2, b2)


# speedup vs baseline: 100.0856x; 100.0856x over previous
import math

import jax
import jax.numpy as jnp
from jax.experimental import pallas as pl
from jax.experimental.pallas import tpu as pltpu


def _moe_body(idx_ref, hw_ref, x_ref, W1_ref, b1_ref, W2_ref, b2_ref, out_ref):
    e = pl.program_id(1)
    x = x_ref[...]
    h = jnp.dot(x, W1_ref[0], preferred_element_type=jnp.float32) + b1_ref[0]
    h = 0.5 * h * (1.0 + jax.lax.erf(h * (1.0 / math.sqrt(2.0))))
    y = jnp.dot(h, W2_ref[0], preferred_element_type=jnp.float32) + b2_ref[0]
    w = jnp.sum(jnp.where(idx_ref[...] == e, hw_ref[...], 0.0), axis=1)
    contrib = w[:, None] * y

    @pl.when(e == 0)
    def _():
        out_ref[...] = contrib

    @pl.when(e != 0)
    def _():
        out_ref[...] += contrib


def kernel(x_modality, expert_indices, hard_weights, W1, b1, W2, b2):
    B, D = x_modality.shape
    E, _, H = W1.shape
    K = expert_indices.shape[1]
    idx = expert_indices.astype(jnp.int32)
    b1r = b1.reshape(E, 1, H)
    b2r = b2.reshape(E, 1, D)

    BM = 512
    nb = B // BM
    grid = (nb, E)

    return pl.pallas_call(
        _moe_body,
        grid=grid,
        in_specs=[
            pl.BlockSpec((BM, K), lambda i, e: (i, 0)),
            pl.BlockSpec((BM, K), lambda i, e: (i, 0)),
            pl.BlockSpec((BM, D), lambda i, e: (i, 0)),
            pl.BlockSpec((1, D, H), lambda i, e: (e, 0, 0)),
            pl.BlockSpec((1, 1, H), lambda i, e: (e, 0, 0)),
            pl.BlockSpec((1, H, D), lambda i, e: (e, 0, 0)),
            pl.BlockSpec((1, 1, D), lambda i, e: (e, 0, 0)),
        ],
        out_specs=pl.BlockSpec((BM, D), lambda i, e: (i, 0)),
        out_shape=jax.ShapeDtypeStruct((B, D), jnp.float32),
        compiler_params=pltpu.CompilerParams(
            dimension_semantics=("parallel", "arbitrary"),
        ),
    )(idx, hard_weights, x_modality, W1, b1r, W2, b2r)


# speedup vs baseline: 120.2029x; 1.2010x over previous
import math

import jax
import jax.numpy as jnp
from jax.experimental import pallas as pl
from jax.experimental.pallas import tpu as pltpu


def _moe_body(idx_ref, hw_ref, x_ref, W1_ref, b1_ref, W2_ref, b2_ref, out_ref):
    e = pl.program_id(1)
    x = x_ref[...]
    h = jnp.dot(x, W1_ref[0], preferred_element_type=jnp.float32) + b1_ref[0]
    h = 0.5 * h * (1.0 + jax.lax.erf(h * (1.0 / math.sqrt(2.0))))
    y = jnp.dot(h, W2_ref[0], preferred_element_type=jnp.float32) + b2_ref[0]
    w = jnp.sum(jnp.where(idx_ref[...] == e, hw_ref[...], 0.0), axis=1)
    contrib = w[:, None] * y

    @pl.when(e == 0)
    def _():
        out_ref[...] = contrib

    @pl.when(e != 0)
    def _():
        out_ref[...] += contrib


def kernel(x_modality, expert_indices, hard_weights, W1, b1, W2, b2):
    B, D = x_modality.shape
    E, _, H = W1.shape
    K = expert_indices.shape[1]
    idx = expert_indices.astype(jnp.int32)
    b1r = b1.reshape(E, 1, H)
    b2r = b2.reshape(E, 1, D)

    BM = 1024
    nb = B // BM
    grid = (nb, E)

    return pl.pallas_call(
        _moe_body,
        grid=grid,
        in_specs=[
            pl.BlockSpec((BM, K), lambda i, e: (i, 0)),
            pl.BlockSpec((BM, K), lambda i, e: (i, 0)),
            pl.BlockSpec((BM, D), lambda i, e: (i, 0)),
            pl.BlockSpec((1, D, H), lambda i, e: (e, 0, 0)),
            pl.BlockSpec((1, 1, H), lambda i, e: (e, 0, 0)),
            pl.BlockSpec((1, H, D), lambda i, e: (e, 0, 0)),
            pl.BlockSpec((1, 1, D), lambda i, e: (e, 0, 0)),
        ],
        out_specs=pl.BlockSpec((BM, D), lambda i, e: (i, 0)),
        out_shape=jax.ShapeDtypeStruct((B, D), jnp.float32),
        compiler_params=pltpu.CompilerParams(
            dimension_semantics=("parallel", "arbitrary"),
        ),
    )(idx, hard_weights, x_modality, W1, b1r, W2, b2r)
